# baseline (speedup 1.0000x reference)
import functools

import jax
import jax.numpy as jnp
from jax.experimental import pallas as pl
from jax.experimental.pallas import tpu as pltpu

_B, _H, _W, _M = 4, 512, 512, 8
_HW = _H * _W
_LN = 4096
_SR = 8
_P = _SR * _LN
_RI = _P // _W
_NJ = _HW // _P


def _tc_body(acc_ref, box_ref, w1t_ref, b1_ref, w2t_ref, b2_ref, x_ref,
             out_ref, *, bstatic):
    del acc_ref
    j = pl.program_id(0)
    w1t = w1t_ref[...]
    b1 = b1_ref[...]
    w2t = w2t_ref[...]
    b2 = b2_ref[...]
    preds = []
    for r in range(_SR):
        xtr = x_ref[:, 0, r, :]
        ht = jax.lax.dot_general(
            w1t, xtr, (((1,), (0,)), ((), ())),
            preferred_element_type=jnp.float32)
        ht = jnp.maximum(ht + b1, 0.0)
        lt = jax.lax.dot_general(
            w2t, ht, (((1,), (0,)), ((), ())),
            preferred_element_type=jnp.float32)
        lt = lt + b2
        preds.append((lt[1:2, :] > lt[0:1, :]).astype(jnp.float32))
    pred1 = jnp.concatenate(preds, axis=0) > 0.5

    n = (j * _P
         + jax.lax.broadcasted_iota(jnp.int32, (_SR, _LN), 0) * _LN
         + jax.lax.broadcasted_iota(jnp.int32, (_SR, _LN), 1))
    v = n >> 9
    u = n & 511
    lab = jnp.ones((_SR, _LN), jnp.float32)
    for m in range(_M):
        x1 = box_ref[bstatic, m, 0]
        y1 = box_ref[bstatic, m, 1]
        x2 = box_ref[bstatic, m, 2]
        y2 = box_ref[bstatic, m, 3]
        lv = box_ref[bstatic, m, 4].astype(jnp.float32)
        mask = (v >= x1) & (v <= x2) & (u >= y1) & (u <= y2) & pred1
        lab = jnp.where(mask, lv, lab)
    out_ref[0] = lab.reshape(_RI, _W)


def _tc_batch(bstatic, acc, xt4, boxi, W1t, b1c, W2t, b2c):
    return pl.pallas_call(
        functools.partial(_tc_body, bstatic=bstatic),
        grid=(_NJ,),
        in_specs=[
            pl.BlockSpec(memory_space=pl.ANY),
            pl.BlockSpec(memory_space=pltpu.SMEM),
            pl.BlockSpec((64, 3), lambda jj: (0, 0)),
            pl.BlockSpec((64, 1), lambda jj: (0, 0)),
            pl.BlockSpec((2, 64), lambda jj: (0, 0)),
            pl.BlockSpec((2, 1), lambda jj: (0, 0)),
            pl.BlockSpec((3, 1, _SR, _LN), lambda jj: (0, jj, 0, 0)),
        ],
        out_specs=pl.BlockSpec((1, _RI, _W), lambda jj: (bstatic, jj, 0)),
        out_shape=jax.ShapeDtypeStruct((_B, _H, _W), jnp.float32),
        input_output_aliases={0: 0},
    )(acc, boxi, W1t, b1c, W2t, b2c, xt4)


def kernel(rgb, depth, intrinsic, box, W1, b1, W2, b2):
    del depth, intrinsic
    boxi = box.astype(jnp.int32)
    W1t = W1.T
    b1c = b1.reshape(64, 1)
    W2t = W2.T
    b2c = b2.reshape(2, 1)
    acc = jnp.zeros((_B, _H, _W), jnp.float32)
    for b in range(_B):
        xt4 = rgb[b].reshape(-1, 3).T.reshape(3, _NJ, _SR, _LN)
        acc = _tc_batch(b, acc, xt4, boxi, W1t, b1c, W2t, b2c)
    return acc


# speedup vs baseline: 1.4928x; 1.4928x over previous
import jax
import jax.numpy as jnp
from jax.experimental import pallas as pl
from jax.experimental.pallas import tpu as pltpu

_B, _H, _W, _M = 4, 512, 512, 8
_RI = 64
_NJ = _H // _RI
_RG = 8


def _tc_body(box_ref, w1t_ref, b1_ref, w2t_ref, b2_ref, x_ref, out_ref,
             pred_ref):
    bidx = pl.program_id(0)
    j = pl.program_id(1)
    w1t = w1t_ref[...]
    b1 = b1_ref[...]
    w2t = w2t_ref[...]
    b2 = b2_ref[...]
    for g in range(_RI // _RG):
        hts = []
        for rr in range(_RG):
            xtr = x_ref[0, :, g * _RG + rr, :]
            ht = jax.lax.dot_general(
                w1t, xtr, (((1,), (0,)), ((), ())),
                preferred_element_type=jnp.float32)
            hts.append(jnp.maximum(ht + b1, 0.0))
        for rr in range(_RG):
            lt = jax.lax.dot_general(
                w2t, hts[rr], (((1,), (0,)), ((), ())),
                preferred_element_type=jnp.float32)
            lt = lt + b2
            pred_ref[pl.ds(g * _RG + rr, 1), :] = (
                lt[1:2, :] > lt[0:1, :]).astype(jnp.float32)

    pred1 = pred_ref[...] > 0.5
    v = j * _RI + jax.lax.broadcasted_iota(jnp.int32, (_RI, _W), 0)
    u = jax.lax.broadcasted_iota(jnp.int32, (_RI, _W), 1)
    lab = jnp.ones((_RI, _W), jnp.float32)
    for m in range(_M):
        x1 = box_ref[bidx, m, 0]
        y1 = box_ref[bidx, m, 1]
        x2 = box_ref[bidx, m, 2]
        y2 = box_ref[bidx, m, 3]
        lv = box_ref[bidx, m, 4].astype(jnp.float32)
        mask = (v >= x1) & (v <= x2) & (u >= y1) & (u <= y2) & pred1
        lab = jnp.where(mask, lv, lab)
    out_ref[0] = lab


def kernel(rgb, depth, intrinsic, box, W1, b1, W2, b2):
    del depth, intrinsic
    rgbp = jnp.transpose(rgb, (0, 3, 1, 2))
    boxi = box.astype(jnp.int32)
    return pl.pallas_call(
        _tc_body,
        grid=(_B, _NJ),
        in_specs=[
            pl.BlockSpec(memory_space=pltpu.SMEM),
            pl.BlockSpec((64, 3), lambda b_, jj: (0, 0)),
            pl.BlockSpec((64, 1), lambda b_, jj: (0, 0)),
            pl.BlockSpec((2, 64), lambda b_, jj: (0, 0)),
            pl.BlockSpec((2, 1), lambda b_, jj: (0, 0)),
            pl.BlockSpec((1, 3, _RI, _W), lambda b_, jj: (b_, 0, jj, 0)),
        ],
        out_specs=pl.BlockSpec((1, _RI, _W), lambda b_, jj: (b_, jj, 0)),
        out_shape=jax.ShapeDtypeStruct((_B, _H, _W), jnp.float32),
        scratch_shapes=[pltpu.VMEM((_RI, _W), jnp.float32)],
    )(boxi, W1.T, b1.reshape(64, 1), W2.T, b2.reshape(2, 1), rgbp)


# speedup vs baseline: 1.5525x; 1.0400x over previous
import jax
import jax.numpy as jnp
from jax.experimental import pallas as pl
from jax.experimental.pallas import tpu as pltpu

_B, _H, _W, _M = 4, 512, 512, 8
_RI = 128
_NJ = _H // _RI
_RG = 8


def _tc_body(box_ref, w1t_ref, b1_ref, w2t_ref, b2_ref, x_ref, out_ref,
             pred_ref):
    bidx = pl.program_id(0)
    j = pl.program_id(1)
    w1t = w1t_ref[...]
    b1 = b1_ref[...]
    w2t = w2t_ref[...]
    b2 = b2_ref[...]
    for g in range(_RI // _RG):
        hts = []
        for rr in range(_RG):
            xtr = x_ref[0, :, g * _RG + rr, :]
            ht = jax.lax.dot_general(
                w1t, xtr, (((1,), (0,)), ((), ())),
                preferred_element_type=jnp.float32)
            hts.append(jnp.maximum(ht + b1, 0.0))
        for rr in range(_RG):
            lt = jax.lax.dot_general(
                w2t, hts[rr], (((1,), (0,)), ((), ())),
                preferred_element_type=jnp.float32)
            lt = lt + b2
            pred_ref[pl.ds(g * _RG + rr, 1), :] = (
                lt[1:2, :] > lt[0:1, :]).astype(jnp.float32)

    pred1 = pred_ref[...] > 0.5
    v = j * _RI + jax.lax.broadcasted_iota(jnp.int32, (_RI, _W), 0)
    u = jax.lax.broadcasted_iota(jnp.int32, (_RI, _W), 1)
    lab = jnp.ones((_RI, _W), jnp.float32)
    for m in range(_M):
        x1 = box_ref[bidx, m, 0]
        y1 = box_ref[bidx, m, 1]
        x2 = box_ref[bidx, m, 2]
        y2 = box_ref[bidx, m, 3]
        lv = box_ref[bidx, m, 4].astype(jnp.float32)
        mask = (v >= x1) & (v <= x2) & (u >= y1) & (u <= y2) & pred1
        lab = jnp.where(mask, lv, lab)
    out_ref[0] = lab


def kernel(rgb, depth, intrinsic, box, W1, b1, W2, b2):
    del depth, intrinsic
    rgbp = jnp.transpose(rgb, (0, 3, 1, 2))
    boxi = box.astype(jnp.int32)
    return pl.pallas_call(
        _tc_body,
        grid=(_B, _NJ),
        in_specs=[
            pl.BlockSpec(memory_space=pltpu.SMEM),
            pl.BlockSpec((64, 3), lambda b_, jj: (0, 0)),
            pl.BlockSpec((64, 1), lambda b_, jj: (0, 0)),
            pl.BlockSpec((2, 64), lambda b_, jj: (0, 0)),
            pl.BlockSpec((2, 1), lambda b_, jj: (0, 0)),
            pl.BlockSpec((1, 3, _RI, _W), lambda b_, jj: (b_, 0, jj, 0)),
        ],
        out_specs=pl.BlockSpec((1, _RI, _W), lambda b_, jj: (b_, jj, 0)),
        out_shape=jax.ShapeDtypeStruct((_B, _H, _W), jnp.float32),
        scratch_shapes=[pltpu.VMEM((_RI, _W), jnp.float32)],
    )(boxi, W1.T, b1.reshape(64, 1), W2.T, b2.reshape(2, 1), rgbp)
